# baseline (speedup 1.0000x reference)
import functools

import jax
import jax.numpy as jnp
from jax import lax
from jax.experimental import pallas as pl
from jax.experimental.pallas import tpu as pltpu
from jax.experimental.pallas import tpu_sc as plsc

N = 10000
D = 128
DH = 64
E = 320000

NC = 2
NS = 16
NW = NC * NS
E_PAD = 327680
K = 4
CH = K * 128
IRPW = E_PAD // NW // 128
RPW = E_PAD // NS // 128
NC2 = RPW // K // 2
N_PAD = 10240
RPT = N_PAD // NS

BM = 1000


def _build_sc():
  out_type = jax.ShapeDtypeStruct((N_PAD, DH), jnp.float32)
  scratch = [
      pltpu.VMEM((K, 128), jnp.int32),
      pltpu.VMEM((K, 128), jnp.int32),
      pltpu.VMEM((K, 128), jnp.int32),
      pltpu.VMEM((K, 128), jnp.int32),
      pltpu.VMEM((CH, DH), jnp.float32),
      pltpu.VMEM((CH, DH), jnp.float32),
      pltpu.VMEM_SHARED((N_PAD, DH), jnp.float32),
      pltpu.SemaphoreType.DMA,
      pltpu.SemaphoreType.DMA,
  ]

  def body(xp, src2d, dst2d, za, out,
           sidx0, didx0, sidx1, didx1, rows0, rows1, acc, sem0, sem1):
    cid = lax.axis_index("c")
    sid = lax.axis_index("s")

    @pl.when(cid == 1)
    def _run():
      pltpu.sync_copy(za, acc.at[pl.ds(sid * RPT, RPT)])
      plsc.subcore_barrier()

      rb0 = sid * RPW

      def fire(c, si, di, buf, sem):
        rb = rb0 + c * K
        pltpu.sync_copy(src2d.at[pl.ds(rb, K)], si)
        pltpu.sync_copy(dst2d.at[pl.ds(rb, K)], di)
        for j in range(K):
          pltpu.async_copy(xp.at[si.at[j]],
                           buf.at[pl.ds(j * 128, 128)], sem)

      def drain(buf, sem):
        pltpu.make_async_copy(xp.at[pl.ds(0, CH)], buf, sem).wait()

      def scatter(di, buf):
        for j in range(K):
          pltpu.sync_copy(buf.at[pl.ds(j * 128, 128)],
                          acc.at[di.at[j]], add=True)

      fire(0, sidx0, didx0, rows0, sem0)

      def step(i, carry):
        c0 = 2 * i
        fire(c0 + 1, sidx1, didx1, rows1, sem1)
        drain(rows0, sem0)
        scatter(didx0, rows0)

        @pl.when(i < NC2 - 1)
        def _():
          fire(c0 + 2, sidx0, didx0, rows0, sem0)

        drain(rows1, sem1)
        scatter(didx1, rows1)
        return carry

      lax.fori_loop(0, NC2, step, 0)
      plsc.subcore_barrier()

      base = sid * RPT
      pltpu.sync_copy(acc.at[pl.ds(base, RPT)],
                      out.at[pl.ds(base, RPT)])

  mesh = plsc.VectorSubcoreMesh(core_axis_name="c", subcore_axis_name="s")
  return pl.kernel(body, out_type=out_type,
                   mesh=mesh, scratch_types=scratch,
                   compiler_params=pltpu.CompilerParams(
                       use_tc_tiling_on_sc=False))


def _build_sc_cnt():
  out_type = jax.ShapeDtypeStruct((NC, N_PAD, 16), jnp.float32)
  scratch = [
      pltpu.VMEM((IRPW, 128), jnp.int32),
      pltpu.VMEM((128, 16), jnp.float32),
      pltpu.VMEM_SHARED((N_PAD, 16), jnp.float32),
  ]

  def body(dst2d, zc, onesc, cnt_out, didx, ones_v, cacc):
    cid = lax.axis_index("c")
    sid = lax.axis_index("s")
    wid = sid * NC + cid
    pltpu.sync_copy(zc, cacc.at[pl.ds(sid * RPT, RPT)])
    pltpu.sync_copy(dst2d.at[pl.ds(wid * IRPW, IRPW)], didx)
    pltpu.sync_copy(onesc, ones_v)
    plsc.subcore_barrier()

    def step(b, carry):
      pltpu.sync_copy(ones_v, cacc.at[didx.at[b]], add=True)
      return carry

    lax.fori_loop(0, IRPW, step, 0)
    plsc.subcore_barrier()
    base = sid * RPT
    pltpu.sync_copy(cacc.at[pl.ds(base, RPT)],
                    cnt_out.at[cid, pl.ds(base, RPT)])

  mesh = plsc.VectorSubcoreMesh(core_axis_name="c", subcore_axis_name="s")
  return pl.kernel(body, out_type=out_type,
                   mesh=mesh, scratch_types=scratch,
                   compiler_params=pltpu.CompilerParams(
                       use_tc_tiling_on_sc=False))


_sc_agg = _build_sc()
_sc_cnt = _build_sc_cnt()


def _proj_body(x_ref, w_ref, b_ref, oa_ref, ob_ref):
  t = jnp.maximum(
      jnp.dot(x_ref[...], w_ref[...], preferred_element_type=jnp.float32)
      + b_ref[...], 0.0)
  oa_ref[...] = t[:, :DH]
  ob_ref[...] = t[:, DH:]


def _proj(x, W, b):
  half = pl.BlockSpec((BM, DH), lambda i: (i, 0))
  return pl.pallas_call(
      _proj_body,
      grid=(N // BM,),
      in_specs=[
          pl.BlockSpec((BM, D), lambda i: (i, 0)),
          pl.BlockSpec((D, D), lambda i: (0, 0)),
          pl.BlockSpec((1, D), lambda i: (0, 0)),
      ],
      out_specs=(half, half),
      out_shape=(jax.ShapeDtypeStruct((N, DH), jnp.float32),
                 jax.ShapeDtypeStruct((N, DH), jnp.float32)),
  )(x, W, b.reshape(1, D))


def _combine(pa, pb, c0, c1, wla, wlb, bl, xr, wr):
  cnt = jnp.maximum(c0[:, 0:1] + c1[:, 0:1], 1.0)
  ma = pa[...] / cnt
  mb = pb[...] / cnt
  return (jnp.dot(ma, wla[...], preferred_element_type=jnp.float32)
          + jnp.dot(mb, wlb[...], preferred_element_type=jnp.float32)
          + jnp.dot(xr[...], wr[...], preferred_element_type=jnp.float32)
          + bl[...])


def _mid_body(pa, pb, c0, c1, x, wla, wlb, bl, wr, wp2, bp2,
              h_ref, xa_ref, xb_ref):
  t = _combine(pa, pb, c0, c1, wla, wlb, bl, x, wr)
  nrm = jnp.sqrt(jnp.sum(t * t, axis=-1, keepdims=True))
  h = jnp.maximum(t / jnp.maximum(nrm, 1e-12), 0.0)
  h_ref[...] = h
  xp2 = jnp.maximum(
      jnp.dot(h, wp2[...], preferred_element_type=jnp.float32) + bp2[...],
      0.0)
  xa_ref[...] = xp2[:, :DH]
  xb_ref[...] = xp2[:, DH:]


def _mid(pa, pb, c0, c1, x, Wl, bl, Wr, Wp2, bp2):
  row = pl.BlockSpec((BM, D), lambda i: (i, 0))
  half = pl.BlockSpec((BM, DH), lambda i: (i, 0))
  cntb = pl.BlockSpec((BM, 16), lambda i: (i, 0))
  whalf = pl.BlockSpec((DH, D), lambda i: (0, 0))
  wspec = pl.BlockSpec((D, D), lambda i: (0, 0))
  bspec = pl.BlockSpec((1, D), lambda i: (0, 0))
  return pl.pallas_call(
      _mid_body,
      grid=(N // BM,),
      in_specs=[half, half, cntb, cntb, row,
                whalf, whalf, bspec, wspec, wspec, bspec],
      out_specs=(row, half, half),
      out_shape=(jax.ShapeDtypeStruct((N, D), jnp.float32),
                 jax.ShapeDtypeStruct((N, DH), jnp.float32),
                 jax.ShapeDtypeStruct((N, DH), jnp.float32)),
  )(pa, pb, c0, c1, x, Wl[:DH], Wl[DH:],
    bl.reshape(1, D), Wr, Wp2, bp2.reshape(1, D))


def _post_body(pa, pb, c0, c1, h, wla, wlb, bl, wr, o_ref):
  t = _combine(pa, pb, c0, c1, wla, wlb, bl, h, wr)
  nrm = jnp.sqrt(jnp.sum(t * t, axis=-1, keepdims=True))
  o_ref[...] = t / jnp.maximum(nrm, 1e-12)


def _post(pa, pb, c0, c1, h, Wl, bl, Wr):
  row = pl.BlockSpec((BM, D), lambda i: (i, 0))
  half = pl.BlockSpec((BM, DH), lambda i: (i, 0))
  cntb = pl.BlockSpec((BM, 16), lambda i: (i, 0))
  whalf = pl.BlockSpec((DH, D), lambda i: (0, 0))
  wspec = pl.BlockSpec((D, D), lambda i: (0, 0))
  bspec = pl.BlockSpec((1, D), lambda i: (0, 0))
  return pl.pallas_call(
      _post_body,
      grid=(N // BM,),
      in_specs=[half, half, cntb, cntb, row,
                whalf, whalf, bspec, wspec],
      out_specs=row,
      out_shape=jax.ShapeDtypeStruct((N, D), jnp.float32),
  )(pa, pb, c0, c1, h, Wl[:DH], Wl[DH:],
    bl.reshape(1, D), Wr)


def kernel(node_emb, edge_index, W1p, b1p, W1l, b1l, W1r,
           W2p, b2p, W2l, b2l, W2r):
  pad = E_PAD - E
  src = jnp.concatenate(
      [edge_index[0], jnp.zeros((pad,), jnp.int32)]).reshape(-1, 128)
  dst = jnp.concatenate(
      [edge_index[1], jnp.full((pad,), N, jnp.int32)]).reshape(-1, 128)
  za = jnp.zeros((RPT, DH), jnp.float32)
  zc = jnp.zeros((RPT, 16), jnp.float32)
  onesc = jnp.ones((128, 16), jnp.float32)

  xa, xb = _proj(node_emb, W1p, b1p)
  cnt1 = _sc_cnt(dst, zc, onesc)
  agg1a = _sc_agg(xa, src, dst, za)
  agg1b = _sc_agg(xb, src, dst, za)
  h, x2a, x2b = _mid(agg1a[:N], agg1b[:N],
                     cnt1[0, :N], cnt1[1, :N],
                     node_emb, W1l, b1l, W1r, W2p, b2p)
  agg2a = _sc_agg(x2a, src, dst, za)
  agg2b = _sc_agg(x2b, src, dst, za)
  out = _post(agg2a[:N], agg2b[:N],
              cnt1[0, :N], cnt1[1, :N], h, W2l, b2l, W2r)
  return out


# speedup vs baseline: 1.0573x; 1.0573x over previous
import functools

import jax
import jax.numpy as jnp
from jax import lax
from jax.experimental import pallas as pl
from jax.experimental.pallas import tpu as pltpu
from jax.experimental.pallas import tpu_sc as plsc

N = 10000
D = 128
DH = 64
E = 320000

NC = 2
NS = 16
NW = NC * NS
E_PAD = 327680
K = 4
CH = K * 128
IRPW = E_PAD // NW // 128
RPW = IRPW
NC2 = RPW // K // 2
N_PAD = 10240
RPT = N_PAD // NS

BM = 1000


def _build_sc():
  out_type = (jax.ShapeDtypeStruct((NC, N_PAD, DH), jnp.float32),
              jax.ShapeDtypeStruct((NC, N_PAD, DH), jnp.float32))
  scratch = [
      pltpu.VMEM((K, 128), jnp.int32),
      pltpu.VMEM((K, 128), jnp.int32),
      pltpu.VMEM((K, 128), jnp.int32),
      pltpu.VMEM((K, 128), jnp.int32),
      pltpu.VMEM((CH, DH), jnp.float32),
      pltpu.VMEM((CH, DH), jnp.float32),
      pltpu.VMEM_SHARED((N_PAD, DH), jnp.float32),
      pltpu.SemaphoreType.DMA,
      pltpu.SemaphoreType.DMA,
  ]

  def body(xpa, xpb, src2d, dst2d, za, outa, outb,
           sidx0, didx0, sidx1, didx1, rows0, rows1, acc, sem0, sem1):
    cid = lax.axis_index("c")
    sid = lax.axis_index("s")
    wid = sid * NC + cid
    rb0 = wid * RPW
    base = sid * RPT

    def phase(xp, out):
      pltpu.sync_copy(za, acc.at[pl.ds(base, RPT)])
      plsc.subcore_barrier()

      def fire(c, si, di, buf, sem):
        rb = rb0 + c * K
        pltpu.sync_copy(src2d.at[pl.ds(rb, K)], si)
        pltpu.sync_copy(dst2d.at[pl.ds(rb, K)], di)
        for j in range(K):
          pltpu.async_copy(xp.at[si.at[j]],
                           buf.at[pl.ds(j * 128, 128)], sem)

      def drain(buf, sem):
        pltpu.make_async_copy(xp.at[pl.ds(0, CH)], buf, sem).wait()

      def scatter(di, buf):
        for j in range(K):
          pltpu.sync_copy(buf.at[pl.ds(j * 128, 128)],
                          acc.at[di.at[j]], add=True)

      fire(0, sidx0, didx0, rows0, sem0)

      def step(i, carry):
        c0 = 2 * i
        fire(c0 + 1, sidx1, didx1, rows1, sem1)
        drain(rows0, sem0)
        scatter(didx0, rows0)

        @pl.when(i < NC2 - 1)
        def _():
          fire(c0 + 2, sidx0, didx0, rows0, sem0)

        drain(rows1, sem1)
        scatter(didx1, rows1)
        return carry

      lax.fori_loop(0, NC2, step, 0)
      plsc.subcore_barrier()
      pltpu.sync_copy(acc.at[pl.ds(base, RPT)],
                      out.at[cid, pl.ds(base, RPT)])
      plsc.subcore_barrier()

    phase(xpa, outa)
    phase(xpb, outb)

  mesh = plsc.VectorSubcoreMesh(core_axis_name="c", subcore_axis_name="s")
  return pl.kernel(body, out_type=out_type,
                   mesh=mesh, scratch_types=scratch,
                   compiler_params=pltpu.CompilerParams(
                       use_tc_tiling_on_sc=False))


def _build_sc_cnt():
  out_type = jax.ShapeDtypeStruct((NC, N_PAD, 16), jnp.float32)
  scratch = [
      pltpu.VMEM((IRPW, 128), jnp.int32),
      pltpu.VMEM((128, 16), jnp.float32),
      pltpu.VMEM_SHARED((N_PAD, 16), jnp.float32),
  ]

  def body(dst2d, zc, onesc, cnt_out, didx, ones_v, cacc):
    cid = lax.axis_index("c")
    sid = lax.axis_index("s")
    wid = sid * NC + cid
    pltpu.sync_copy(zc, cacc.at[pl.ds(sid * RPT, RPT)])
    pltpu.sync_copy(dst2d.at[pl.ds(wid * IRPW, IRPW)], didx)
    pltpu.sync_copy(onesc, ones_v)
    plsc.subcore_barrier()

    def step(b, carry):
      pltpu.sync_copy(ones_v, cacc.at[didx.at[b]], add=True)
      return carry

    lax.fori_loop(0, IRPW, step, 0)
    plsc.subcore_barrier()
    base = sid * RPT
    pltpu.sync_copy(cacc.at[pl.ds(base, RPT)],
                    cnt_out.at[cid, pl.ds(base, RPT)])

  mesh = plsc.VectorSubcoreMesh(core_axis_name="c", subcore_axis_name="s")
  return pl.kernel(body, out_type=out_type,
                   mesh=mesh, scratch_types=scratch,
                   compiler_params=pltpu.CompilerParams(
                       use_tc_tiling_on_sc=False))


_sc_agg = _build_sc()
_sc_cnt = _build_sc_cnt()


def _proj_body(x_ref, w_ref, b_ref, oa_ref, ob_ref):
  t = jnp.maximum(
      jnp.dot(x_ref[...], w_ref[...], preferred_element_type=jnp.float32)
      + b_ref[...], 0.0)
  oa_ref[...] = t[:, :DH]
  ob_ref[...] = t[:, DH:]


def _proj(x, W, b):
  half = pl.BlockSpec((BM, DH), lambda i: (i, 0))
  return pl.pallas_call(
      _proj_body,
      grid=(N // BM,),
      in_specs=[
          pl.BlockSpec((BM, D), lambda i: (i, 0)),
          pl.BlockSpec((D, D), lambda i: (0, 0)),
          pl.BlockSpec((1, D), lambda i: (0, 0)),
      ],
      out_specs=(half, half),
      out_shape=(jax.ShapeDtypeStruct((N, DH), jnp.float32),
                 jax.ShapeDtypeStruct((N, DH), jnp.float32)),
  )(x, W, b.reshape(1, D))


def _combine(pa, pb, c0, c1, wla, wlb, bl, xr, wr):
  cnt = jnp.maximum(c0[:, 0:1] + c1[:, 0:1], 1.0)
  ma = (pa[0] + pa[1]) / cnt
  mb = (pb[0] + pb[1]) / cnt
  return (jnp.dot(ma, wla[...], preferred_element_type=jnp.float32)
          + jnp.dot(mb, wlb[...], preferred_element_type=jnp.float32)
          + jnp.dot(xr[...], wr[...], preferred_element_type=jnp.float32)
          + bl[...])


def _mid_body(pa, pb, c0, c1, x, wla, wlb, bl, wr, wp2, bp2,
              h_ref, xa_ref, xb_ref):
  t = _combine(pa, pb, c0, c1, wla, wlb, bl, x, wr)
  nrm = jnp.sqrt(jnp.sum(t * t, axis=-1, keepdims=True))
  h = jnp.maximum(t / jnp.maximum(nrm, 1e-12), 0.0)
  h_ref[...] = h
  xp2 = jnp.maximum(
      jnp.dot(h, wp2[...], preferred_element_type=jnp.float32) + bp2[...],
      0.0)
  xa_ref[...] = xp2[:, :DH]
  xb_ref[...] = xp2[:, DH:]


def _mid(pa, pb, c0, c1, x, Wl, bl, Wr, Wp2, bp2):
  row = pl.BlockSpec((BM, D), lambda i: (i, 0))
  part = pl.BlockSpec((NC, BM, DH), lambda i: (0, i, 0))
  cntb = pl.BlockSpec((BM, 16), lambda i: (i, 0))
  whalf = pl.BlockSpec((DH, D), lambda i: (0, 0))
  wspec = pl.BlockSpec((D, D), lambda i: (0, 0))
  bspec = pl.BlockSpec((1, D), lambda i: (0, 0))
  half = pl.BlockSpec((BM, DH), lambda i: (i, 0))
  return pl.pallas_call(
      _mid_body,
      grid=(N // BM,),
      in_specs=[part, part, cntb, cntb, row,
                whalf, whalf, bspec, wspec, wspec, bspec],
      out_specs=(row, half, half),
      out_shape=(jax.ShapeDtypeStruct((N, D), jnp.float32),
                 jax.ShapeDtypeStruct((N, DH), jnp.float32),
                 jax.ShapeDtypeStruct((N, DH), jnp.float32)),
  )(pa, pb, c0, c1, x, Wl[:DH], Wl[DH:],
    bl.reshape(1, D), Wr, Wp2, bp2.reshape(1, D))


def _post_body(pa, pb, c0, c1, h, wla, wlb, bl, wr, o_ref):
  t = _combine(pa, pb, c0, c1, wla, wlb, bl, h, wr)
  nrm = jnp.sqrt(jnp.sum(t * t, axis=-1, keepdims=True))
  o_ref[...] = t / jnp.maximum(nrm, 1e-12)


def _post(pa, pb, c0, c1, h, Wl, bl, Wr):
  row = pl.BlockSpec((BM, D), lambda i: (i, 0))
  part = pl.BlockSpec((NC, BM, DH), lambda i: (0, i, 0))
  cntb = pl.BlockSpec((BM, 16), lambda i: (i, 0))
  whalf = pl.BlockSpec((DH, D), lambda i: (0, 0))
  wspec = pl.BlockSpec((D, D), lambda i: (0, 0))
  bspec = pl.BlockSpec((1, D), lambda i: (0, 0))
  return pl.pallas_call(
      _post_body,
      grid=(N // BM,),
      in_specs=[part, part, cntb, cntb, row,
                whalf, whalf, bspec, wspec],
      out_specs=row,
      out_shape=jax.ShapeDtypeStruct((N, D), jnp.float32),
  )(pa, pb, c0, c1, h, Wl[:DH], Wl[DH:],
    bl.reshape(1, D), Wr)


def kernel(node_emb, edge_index, W1p, b1p, W1l, b1l, W1r,
           W2p, b2p, W2l, b2l, W2r):
  pad = E_PAD - E
  src = jnp.concatenate(
      [edge_index[0], jnp.zeros((pad,), jnp.int32)]).reshape(-1, 128)
  dst = jnp.concatenate(
      [edge_index[1], jnp.full((pad,), N, jnp.int32)]).reshape(-1, 128)
  za = jnp.zeros((RPT, DH), jnp.float32)
  zc = jnp.zeros((RPT, 16), jnp.float32)
  onesc = jnp.ones((128, 16), jnp.float32)

  xa, xb = _proj(node_emb, W1p, b1p)
  cnt1 = _sc_cnt(dst, zc, onesc)
  agg1a, agg1b = _sc_agg(xa, xb, src, dst, za)
  h, x2a, x2b = _mid(agg1a[:, :N], agg1b[:, :N],
                     cnt1[0, :N], cnt1[1, :N],
                     node_emb, W1l, b1l, W1r, W2p, b2p)
  agg2a, agg2b = _sc_agg(x2a, x2b, src, dst, za)
  out = _post(agg2a[:, :N], agg2b[:, :N],
              cnt1[0, :N], cnt1[1, :N], h, W2l, b2l, W2r)
  return out


# speedup vs baseline: 1.1003x; 1.0407x over previous
import functools

import jax
import jax.numpy as jnp
from jax import lax
from jax.experimental import pallas as pl
from jax.experimental.pallas import tpu as pltpu
from jax.experimental.pallas import tpu_sc as plsc

N = 10000
D = 128
DH = 64
E = 320000

NC = 2
NS = 16
NW = NC * NS
E_PAD = 327680
K = 4
CH = K * 128
IRPW = E_PAD // NW // 128
R0 = 120
R1 = 40
NC2_0 = R0 // K // 2
NC2_1 = R1 // K // 2
N_PAD = 10240
RPT = N_PAD // NS

BM = 1000


def _build_sc():
  out_type = (jax.ShapeDtypeStruct((NC, N_PAD, DH), jnp.float32),
              jax.ShapeDtypeStruct((NC, N_PAD, DH), jnp.float32))
  scratch = [
      pltpu.VMEM((K, 128), jnp.int32),
      pltpu.VMEM((K, 128), jnp.int32),
      pltpu.VMEM((K, 128), jnp.int32),
      pltpu.VMEM((K, 128), jnp.int32),
      pltpu.VMEM((CH, DH), jnp.float32),
      pltpu.VMEM((CH, DH), jnp.float32),
      pltpu.VMEM_SHARED((N_PAD, DH), jnp.float32),
      pltpu.SemaphoreType.DMA,
      pltpu.SemaphoreType.DMA,
  ]

  def body(xpa, xpb, src2d, dst2d, za, outa, outb,
           sidx0, didx0, sidx1, didx1, rows0, rows1, acc, sem0, sem1):
    cid = lax.axis_index("c")
    sid = lax.axis_index("s")
    rb0 = jnp.where(cid == 0, sid * R0, NS * R0 + sid * R1)
    nc2 = jnp.where(cid == 0, NC2_0, NC2_1)
    base = sid * RPT

    def phase(xp, out):
      pltpu.sync_copy(za, acc.at[pl.ds(base, RPT)])
      plsc.subcore_barrier()

      def fire(c, si, di, buf, sem):
        rb = rb0 + c * K
        pltpu.sync_copy(src2d.at[pl.ds(rb, K)], si)
        pltpu.sync_copy(dst2d.at[pl.ds(rb, K)], di)
        for j in range(K):
          pltpu.async_copy(xp.at[si.at[j]],
                           buf.at[pl.ds(j * 128, 128)], sem)

      def drain(buf, sem):
        pltpu.make_async_copy(xp.at[pl.ds(0, CH)], buf, sem).wait()

      def scatter(di, buf):
        for j in range(K):
          pltpu.sync_copy(buf.at[pl.ds(j * 128, 128)],
                          acc.at[di.at[j]], add=True)

      fire(0, sidx0, didx0, rows0, sem0)

      def step(i, carry):
        c0 = 2 * i
        fire(c0 + 1, sidx1, didx1, rows1, sem1)
        drain(rows0, sem0)
        scatter(didx0, rows0)

        @pl.when(i < nc2 - 1)
        def _():
          fire(c0 + 2, sidx0, didx0, rows0, sem0)

        drain(rows1, sem1)
        scatter(didx1, rows1)
        return carry

      lax.fori_loop(0, nc2, step, 0)
      plsc.subcore_barrier()
      pltpu.sync_copy(acc.at[pl.ds(base, RPT)],
                      out.at[cid, pl.ds(base, RPT)])
      plsc.subcore_barrier()

    phase(xpa, outa)
    phase(xpb, outb)

  mesh = plsc.VectorSubcoreMesh(core_axis_name="c", subcore_axis_name="s")
  return pl.kernel(body, out_type=out_type,
                   mesh=mesh, scratch_types=scratch,
                   compiler_params=pltpu.CompilerParams(
                       use_tc_tiling_on_sc=False))


def _build_sc_cnt():
  out_type = jax.ShapeDtypeStruct((NC, N_PAD, 16), jnp.float32)
  scratch = [
      pltpu.VMEM((IRPW, 128), jnp.int32),
      pltpu.VMEM((128, 16), jnp.float32),
      pltpu.VMEM_SHARED((N_PAD, 16), jnp.float32),
  ]

  def body(dst2d, zc, onesc, cnt_out, didx, ones_v, cacc):
    cid = lax.axis_index("c")
    sid = lax.axis_index("s")
    wid = sid * NC + cid
    pltpu.sync_copy(zc, cacc.at[pl.ds(sid * RPT, RPT)])
    pltpu.sync_copy(dst2d.at[pl.ds(wid * IRPW, IRPW)], didx)
    pltpu.sync_copy(onesc, ones_v)
    plsc.subcore_barrier()

    def step(b, carry):
      pltpu.sync_copy(ones_v, cacc.at[didx.at[b]], add=True)
      return carry

    lax.fori_loop(0, IRPW, step, 0)
    plsc.subcore_barrier()
    base = sid * RPT
    pltpu.sync_copy(cacc.at[pl.ds(base, RPT)],
                    cnt_out.at[cid, pl.ds(base, RPT)])

  mesh = plsc.VectorSubcoreMesh(core_axis_name="c", subcore_axis_name="s")
  return pl.kernel(body, out_type=out_type,
                   mesh=mesh, scratch_types=scratch,
                   compiler_params=pltpu.CompilerParams(
                       use_tc_tiling_on_sc=False))


_sc_agg = _build_sc()
_sc_cnt = _build_sc_cnt()


def _proj_body(x_ref, w_ref, b_ref, oa_ref, ob_ref):
  t = jnp.maximum(
      jnp.dot(x_ref[...], w_ref[...], preferred_element_type=jnp.float32)
      + b_ref[...], 0.0)
  oa_ref[...] = t[:, :DH]
  ob_ref[...] = t[:, DH:]


def _proj(x, W, b):
  half = pl.BlockSpec((BM, DH), lambda i: (i, 0))
  return pl.pallas_call(
      _proj_body,
      grid=(N // BM,),
      in_specs=[
          pl.BlockSpec((BM, D), lambda i: (i, 0)),
          pl.BlockSpec((D, D), lambda i: (0, 0)),
          pl.BlockSpec((1, D), lambda i: (0, 0)),
      ],
      out_specs=(half, half),
      out_shape=(jax.ShapeDtypeStruct((N, DH), jnp.float32),
                 jax.ShapeDtypeStruct((N, DH), jnp.float32)),
  )(x, W, b.reshape(1, D))


def _combine(pa, pb, c0, c1, wla, wlb, bl, xr, wr):
  cnt = jnp.maximum(c0[:, 0:1] + c1[:, 0:1], 1.0)
  ma = (pa[0] + pa[1]) / cnt
  mb = (pb[0] + pb[1]) / cnt
  return (jnp.dot(ma, wla[...], preferred_element_type=jnp.float32)
          + jnp.dot(mb, wlb[...], preferred_element_type=jnp.float32)
          + jnp.dot(xr[...], wr[...], preferred_element_type=jnp.float32)
          + bl[...])


def _mid_body(pa, pb, c0, c1, x, wla, wlb, bl, wr, wp2, bp2,
              h_ref, xa_ref, xb_ref):
  t = _combine(pa, pb, c0, c1, wla, wlb, bl, x, wr)
  nrm = jnp.sqrt(jnp.sum(t * t, axis=-1, keepdims=True))
  h = jnp.maximum(t / jnp.maximum(nrm, 1e-12), 0.0)
  h_ref[...] = h
  xp2 = jnp.maximum(
      jnp.dot(h, wp2[...], preferred_element_type=jnp.float32) + bp2[...],
      0.0)
  xa_ref[...] = xp2[:, :DH]
  xb_ref[...] = xp2[:, DH:]


def _mid(pa, pb, c0, c1, x, Wl, bl, Wr, Wp2, bp2):
  row = pl.BlockSpec((BM, D), lambda i: (i, 0))
  part = pl.BlockSpec((NC, BM, DH), lambda i: (0, i, 0))
  cntb = pl.BlockSpec((BM, 16), lambda i: (i, 0))
  whalf = pl.BlockSpec((DH, D), lambda i: (0, 0))
  wspec = pl.BlockSpec((D, D), lambda i: (0, 0))
  bspec = pl.BlockSpec((1, D), lambda i: (0, 0))
  half = pl.BlockSpec((BM, DH), lambda i: (i, 0))
  return pl.pallas_call(
      _mid_body,
      grid=(N // BM,),
      in_specs=[part, part, cntb, cntb, row,
                whalf, whalf, bspec, wspec, wspec, bspec],
      out_specs=(row, half, half),
      out_shape=(jax.ShapeDtypeStruct((N, D), jnp.float32),
                 jax.ShapeDtypeStruct((N, DH), jnp.float32),
                 jax.ShapeDtypeStruct((N, DH), jnp.float32)),
  )(pa, pb, c0, c1, x, Wl[:DH], Wl[DH:],
    bl.reshape(1, D), Wr, Wp2, bp2.reshape(1, D))


def _post_body(pa, pb, c0, c1, h, wla, wlb, bl, wr, o_ref):
  t = _combine(pa, pb, c0, c1, wla, wlb, bl, h, wr)
  nrm = jnp.sqrt(jnp.sum(t * t, axis=-1, keepdims=True))
  o_ref[...] = t / jnp.maximum(nrm, 1e-12)


def _post(pa, pb, c0, c1, h, Wl, bl, Wr):
  row = pl.BlockSpec((BM, D), lambda i: (i, 0))
  part = pl.BlockSpec((NC, BM, DH), lambda i: (0, i, 0))
  cntb = pl.BlockSpec((BM, 16), lambda i: (i, 0))
  whalf = pl.BlockSpec((DH, D), lambda i: (0, 0))
  wspec = pl.BlockSpec((D, D), lambda i: (0, 0))
  bspec = pl.BlockSpec((1, D), lambda i: (0, 0))
  return pl.pallas_call(
      _post_body,
      grid=(N // BM,),
      in_specs=[part, part, cntb, cntb, row,
                whalf, whalf, bspec, wspec],
      out_specs=row,
      out_shape=jax.ShapeDtypeStruct((N, D), jnp.float32),
  )(pa, pb, c0, c1, h, Wl[:DH], Wl[DH:],
    bl.reshape(1, D), Wr)


def kernel(node_emb, edge_index, W1p, b1p, W1l, b1l, W1r,
           W2p, b2p, W2l, b2l, W2r):
  pad = E_PAD - E
  src = jnp.concatenate(
      [edge_index[0], jnp.zeros((pad,), jnp.int32)]).reshape(-1, 128)
  dst = jnp.concatenate(
      [edge_index[1], jnp.full((pad,), N, jnp.int32)]).reshape(-1, 128)
  za = jnp.zeros((RPT, DH), jnp.float32)
  zc = jnp.zeros((RPT, 16), jnp.float32)
  onesc = jnp.ones((128, 16), jnp.float32)

  xa, xb = _proj(node_emb, W1p, b1p)
  cnt1 = _sc_cnt(dst, zc, onesc)
  agg1a, agg1b = _sc_agg(xa, xb, src, dst, za)
  h, x2a, x2b = _mid(agg1a[:, :N], agg1b[:, :N],
                     cnt1[0, :N], cnt1[1, :N],
                     node_emb, W1l, b1l, W1r, W2p, b2p)
  agg2a, agg2b = _sc_agg(x2a, x2b, src, dst, za)
  out = _post(agg2a[:, :N], agg2b[:, :N],
              cnt1[0, :N], cnt1[1, :N], h, W2l, b2l, W2r)
  return out


# speedup vs baseline: 1.2568x; 1.1423x over previous
import functools

import jax
import jax.numpy as jnp
from jax import lax
from jax.experimental import pallas as pl
from jax.experimental.pallas import tpu as pltpu
from jax.experimental.pallas import tpu_sc as plsc

N = 10000
D = 128
DH = 64
E = 320000

NC = 2
NS = 16
NW = NC * NS
E_PAD = 327680
K = 4
CH = K * 128
IRPW = E_PAD // NW // 128
R0 = 144
R1 = 16
NC2_0 = R0 // K // 2
NC2_1 = R1 // K // 2
N_PAD = 10240
RPT = N_PAD // NS

BM = 1000


def _build_sc():
  out_type = (jax.ShapeDtypeStruct((NC, N_PAD, DH), jnp.float32),
              jax.ShapeDtypeStruct((NC, N_PAD, DH), jnp.float32))
  scratch = [
      pltpu.VMEM((K, 128), jnp.int32),
      pltpu.VMEM((K, 128), jnp.int32),
      pltpu.VMEM((K, 128), jnp.int32),
      pltpu.VMEM((K, 128), jnp.int32),
      pltpu.VMEM((CH, DH), jnp.float32),
      pltpu.VMEM((CH, DH), jnp.float32),
      pltpu.VMEM_SHARED((N_PAD, DH), jnp.float32),
      pltpu.SemaphoreType.DMA,
      pltpu.SemaphoreType.DMA,
  ]

  def body(xpa, xpb, src2d, dst2d, za, outa, outb,
           sidx0, didx0, sidx1, didx1, rows0, rows1, acc, sem0, sem1):
    cid = lax.axis_index("c")
    sid = lax.axis_index("s")
    rb0 = jnp.where(cid == 0, sid * R0, NS * R0 + sid * R1)
    nc2 = jnp.where(cid == 0, NC2_0, NC2_1)
    base = sid * RPT

    def phase(xp, out):
      pltpu.sync_copy(za, acc.at[pl.ds(base, RPT)])
      plsc.subcore_barrier()

      def fire(c, si, di, buf, sem):
        rb = rb0 + c * K
        pltpu.sync_copy(src2d.at[pl.ds(rb, K)], si)
        pltpu.sync_copy(dst2d.at[pl.ds(rb, K)], di)
        for j in range(K):
          pltpu.async_copy(xp.at[si.at[j]],
                           buf.at[pl.ds(j * 128, 128)], sem)

      def drain(buf, sem):
        pltpu.make_async_copy(xp.at[pl.ds(0, CH)], buf, sem).wait()

      def scatter(di, buf):
        for j in range(K):
          pltpu.sync_copy(buf.at[pl.ds(j * 128, 128)],
                          acc.at[di.at[j]], add=True)

      fire(0, sidx0, didx0, rows0, sem0)

      def step(i, carry):
        c0 = 2 * i
        fire(c0 + 1, sidx1, didx1, rows1, sem1)
        drain(rows0, sem0)
        scatter(didx0, rows0)

        @pl.when(i < nc2 - 1)
        def _():
          fire(c0 + 2, sidx0, didx0, rows0, sem0)

        drain(rows1, sem1)
        scatter(didx1, rows1)
        return carry

      lax.fori_loop(0, nc2, step, 0)
      plsc.subcore_barrier()
      pltpu.sync_copy(acc.at[pl.ds(base, RPT)],
                      out.at[cid, pl.ds(base, RPT)])
      plsc.subcore_barrier()

    phase(xpa, outa)
    phase(xpb, outb)

  mesh = plsc.VectorSubcoreMesh(core_axis_name="c", subcore_axis_name="s")
  return pl.kernel(body, out_type=out_type,
                   mesh=mesh, scratch_types=scratch,
                   compiler_params=pltpu.CompilerParams(
                       use_tc_tiling_on_sc=False))


def _build_sc_cnt():
  out_type = jax.ShapeDtypeStruct((NC, N_PAD, 16), jnp.float32)
  scratch = [
      pltpu.VMEM((IRPW, 128), jnp.int32),
      pltpu.VMEM((128, 16), jnp.float32),
      pltpu.VMEM_SHARED((N_PAD, 16), jnp.float32),
  ]

  def body(dst2d, zc, onesc, cnt_out, didx, ones_v, cacc):
    cid = lax.axis_index("c")
    sid = lax.axis_index("s")
    wid = sid * NC + cid
    pltpu.sync_copy(zc, cacc.at[pl.ds(sid * RPT, RPT)])
    pltpu.sync_copy(dst2d.at[pl.ds(wid * IRPW, IRPW)], didx)
    pltpu.sync_copy(onesc, ones_v)
    plsc.subcore_barrier()

    def step(b, carry):
      pltpu.sync_copy(ones_v, cacc.at[didx.at[b]], add=True)
      return carry

    lax.fori_loop(0, IRPW, step, 0)
    plsc.subcore_barrier()
    base = sid * RPT
    pltpu.sync_copy(cacc.at[pl.ds(base, RPT)],
                    cnt_out.at[cid, pl.ds(base, RPT)])

  mesh = plsc.VectorSubcoreMesh(core_axis_name="c", subcore_axis_name="s")
  return pl.kernel(body, out_type=out_type,
                   mesh=mesh, scratch_types=scratch,
                   compiler_params=pltpu.CompilerParams(
                       use_tc_tiling_on_sc=False))


_sc_agg = _build_sc()
_sc_cnt = _build_sc_cnt()


def _proj_body(x_ref, w_ref, b_ref, oa_ref, ob_ref):
  t = jnp.maximum(
      jnp.dot(x_ref[...], w_ref[...], preferred_element_type=jnp.float32)
      + b_ref[...], 0.0)
  oa_ref[...] = t[:, :DH]
  ob_ref[...] = t[:, DH:]


def _proj(x, W, b):
  half = pl.BlockSpec((BM, DH), lambda i: (i, 0))
  return pl.pallas_call(
      _proj_body,
      grid=(N // BM,),
      in_specs=[
          pl.BlockSpec((BM, D), lambda i: (i, 0)),
          pl.BlockSpec((D, D), lambda i: (0, 0)),
          pl.BlockSpec((1, D), lambda i: (0, 0)),
      ],
      out_specs=(half, half),
      out_shape=(jax.ShapeDtypeStruct((N, DH), jnp.float32),
                 jax.ShapeDtypeStruct((N, DH), jnp.float32)),
  )(x, W, b.reshape(1, D))


def _combine(pa, pb, c0, c1, wla, wlb, bl, xr, wr):
  cnt = jnp.maximum(c0[:, 0:1] + c1[:, 0:1], 1.0)
  ma = (pa[0] + pa[1]) / cnt
  mb = (pb[0] + pb[1]) / cnt
  return (jnp.dot(ma, wla[...], preferred_element_type=jnp.float32)
          + jnp.dot(mb, wlb[...], preferred_element_type=jnp.float32)
          + jnp.dot(xr[...], wr[...], preferred_element_type=jnp.float32)
          + bl[...])


def _mid_body(pa, pb, c0, c1, x, wla, wlb, bl, wr, wp2, bp2,
              h_ref, xa_ref, xb_ref):
  t = _combine(pa, pb, c0, c1, wla, wlb, bl, x, wr)
  nrm = jnp.sqrt(jnp.sum(t * t, axis=-1, keepdims=True))
  h = jnp.maximum(t / jnp.maximum(nrm, 1e-12), 0.0)
  h_ref[...] = h
  xp2 = jnp.maximum(
      jnp.dot(h, wp2[...], preferred_element_type=jnp.float32) + bp2[...],
      0.0)
  xa_ref[...] = xp2[:, :DH]
  xb_ref[...] = xp2[:, DH:]


def _mid(pa, pb, c0, c1, x, Wl, bl, Wr, Wp2, bp2):
  row = pl.BlockSpec((BM, D), lambda i: (i, 0))
  part = pl.BlockSpec((NC, BM, DH), lambda i: (0, i, 0))
  cntb = pl.BlockSpec((BM, 16), lambda i: (i, 0))
  whalf = pl.BlockSpec((DH, D), lambda i: (0, 0))
  wspec = pl.BlockSpec((D, D), lambda i: (0, 0))
  bspec = pl.BlockSpec((1, D), lambda i: (0, 0))
  half = pl.BlockSpec((BM, DH), lambda i: (i, 0))
  return pl.pallas_call(
      _mid_body,
      grid=(N // BM,),
      in_specs=[part, part, cntb, cntb, row,
                whalf, whalf, bspec, wspec, wspec, bspec],
      out_specs=(row, half, half),
      out_shape=(jax.ShapeDtypeStruct((N, D), jnp.float32),
                 jax.ShapeDtypeStruct((N, DH), jnp.float32),
                 jax.ShapeDtypeStruct((N, DH), jnp.float32)),
  )(pa, pb, c0, c1, x, Wl[:DH], Wl[DH:],
    bl.reshape(1, D), Wr, Wp2, bp2.reshape(1, D))


def _post_body(pa, pb, c0, c1, h, wla, wlb, bl, wr, o_ref):
  t = _combine(pa, pb, c0, c1, wla, wlb, bl, h, wr)
  nrm = jnp.sqrt(jnp.sum(t * t, axis=-1, keepdims=True))
  o_ref[...] = t / jnp.maximum(nrm, 1e-12)


def _post(pa, pb, c0, c1, h, Wl, bl, Wr):
  row = pl.BlockSpec((BM, D), lambda i: (i, 0))
  part = pl.BlockSpec((NC, BM, DH), lambda i: (0, i, 0))
  cntb = pl.BlockSpec((BM, 16), lambda i: (i, 0))
  whalf = pl.BlockSpec((DH, D), lambda i: (0, 0))
  wspec = pl.BlockSpec((D, D), lambda i: (0, 0))
  bspec = pl.BlockSpec((1, D), lambda i: (0, 0))
  return pl.pallas_call(
      _post_body,
      grid=(N // BM,),
      in_specs=[part, part, cntb, cntb, row,
                whalf, whalf, bspec, wspec],
      out_specs=row,
      out_shape=jax.ShapeDtypeStruct((N, D), jnp.float32),
  )(pa, pb, c0, c1, h, Wl[:DH], Wl[DH:],
    bl.reshape(1, D), Wr)


def kernel(node_emb, edge_index, W1p, b1p, W1l, b1l, W1r,
           W2p, b2p, W2l, b2l, W2r):
  pad = E_PAD - E
  src = jnp.concatenate(
      [edge_index[0], jnp.zeros((pad,), jnp.int32)]).reshape(-1, 128)
  dst = jnp.concatenate(
      [edge_index[1], jnp.full((pad,), N, jnp.int32)]).reshape(-1, 128)
  za = jnp.zeros((RPT, DH), jnp.float32)
  zc = jnp.zeros((RPT, 16), jnp.float32)
  onesc = jnp.ones((128, 16), jnp.float32)

  xa, xb = _proj(node_emb, W1p, b1p)
  cnt1 = _sc_cnt(dst, zc, onesc)
  agg1a, agg1b = _sc_agg(xa, xb, src, dst, za)
  h, x2a, x2b = _mid(agg1a[:, :N], agg1b[:, :N],
                     cnt1[0, :N], cnt1[1, :N],
                     node_emb, W1l, b1l, W1r, W2p, b2p)
  agg2a, agg2b = _sc_agg(x2a, x2b, src, dst, za)
  out = _post(agg2a[:, :N], agg2b[:, :N],
              cnt1[0, :N], cnt1[1, :N], h, W2l, b2l, W2r)
  return out


# speedup vs baseline: 1.2601x; 1.0026x over previous
import functools

import jax
import jax.numpy as jnp
from jax import lax
from jax.experimental import pallas as pl
from jax.experimental.pallas import tpu as pltpu
from jax.experimental.pallas import tpu_sc as plsc

N = 10000
D = 128
DH = 64
E = 320000

NC = 2
NS = 16
NW = NC * NS
E_PAD = 327680
K = 4
CH = K * 128
IRPW = E_PAD // NW // 128
R0 = 152
R1 = 8
NC2_0 = R0 // K // 2
NC2_1 = R1 // K // 2
N_PAD = 10240
RPT = N_PAD // NS

BM = 1000


def _build_sc():
  out_type = (jax.ShapeDtypeStruct((NC, N_PAD, DH), jnp.float32),
              jax.ShapeDtypeStruct((NC, N_PAD, DH), jnp.float32))
  scratch = [
      pltpu.VMEM((K, 128), jnp.int32),
      pltpu.VMEM((K, 128), jnp.int32),
      pltpu.VMEM((K, 128), jnp.int32),
      pltpu.VMEM((K, 128), jnp.int32),
      pltpu.VMEM((CH, DH), jnp.float32),
      pltpu.VMEM((CH, DH), jnp.float32),
      pltpu.VMEM_SHARED((N_PAD, DH), jnp.float32),
      pltpu.SemaphoreType.DMA,
      pltpu.SemaphoreType.DMA,
  ]

  def body(xpa, xpb, src2d, dst2d, za, outa, outb,
           sidx0, didx0, sidx1, didx1, rows0, rows1, acc, sem0, sem1):
    cid = lax.axis_index("c")
    sid = lax.axis_index("s")
    rb0 = jnp.where(cid == 0, sid * R0, NS * R0 + sid * R1)
    nc2 = jnp.where(cid == 0, NC2_0, NC2_1)
    base = sid * RPT

    def phase(xp, out):
      pltpu.sync_copy(za, acc.at[pl.ds(base, RPT)])
      plsc.subcore_barrier()

      def fire(c, si, di, buf, sem):
        rb = rb0 + c * K
        pltpu.sync_copy(src2d.at[pl.ds(rb, K)], si)
        pltpu.sync_copy(dst2d.at[pl.ds(rb, K)], di)
        for j in range(K):
          pltpu.async_copy(xp.at[si.at[j]],
                           buf.at[pl.ds(j * 128, 128)], sem)

      def drain(buf, sem):
        pltpu.make_async_copy(xp.at[pl.ds(0, CH)], buf, sem).wait()

      def scatter(di, buf):
        for j in range(K):
          pltpu.sync_copy(buf.at[pl.ds(j * 128, 128)],
                          acc.at[di.at[j]], add=True)

      fire(0, sidx0, didx0, rows0, sem0)

      def step(i, carry):
        c0 = 2 * i
        fire(c0 + 1, sidx1, didx1, rows1, sem1)
        drain(rows0, sem0)
        scatter(didx0, rows0)

        @pl.when(i < nc2 - 1)
        def _():
          fire(c0 + 2, sidx0, didx0, rows0, sem0)

        drain(rows1, sem1)
        scatter(didx1, rows1)
        return carry

      lax.fori_loop(0, nc2, step, 0)
      plsc.subcore_barrier()
      pltpu.sync_copy(acc.at[pl.ds(base, RPT)],
                      out.at[cid, pl.ds(base, RPT)])
      plsc.subcore_barrier()

    phase(xpa, outa)
    phase(xpb, outb)

  mesh = plsc.VectorSubcoreMesh(core_axis_name="c", subcore_axis_name="s")
  return pl.kernel(body, out_type=out_type,
                   mesh=mesh, scratch_types=scratch,
                   compiler_params=pltpu.CompilerParams(
                       use_tc_tiling_on_sc=False))


def _build_sc_cnt():
  out_type = jax.ShapeDtypeStruct((NC, N_PAD, 16), jnp.float32)
  scratch = [
      pltpu.VMEM((IRPW, 128), jnp.int32),
      pltpu.VMEM((128, 16), jnp.float32),
      pltpu.VMEM_SHARED((N_PAD, 16), jnp.float32),
  ]

  def body(dst2d, zc, onesc, cnt_out, didx, ones_v, cacc):
    cid = lax.axis_index("c")
    sid = lax.axis_index("s")
    wid = sid * NC + cid
    pltpu.sync_copy(zc, cacc.at[pl.ds(sid * RPT, RPT)])
    pltpu.sync_copy(dst2d.at[pl.ds(wid * IRPW, IRPW)], didx)
    pltpu.sync_copy(onesc, ones_v)
    plsc.subcore_barrier()

    def step(b, carry):
      pltpu.sync_copy(ones_v, cacc.at[didx.at[b]], add=True)
      return carry

    lax.fori_loop(0, IRPW, step, 0)
    plsc.subcore_barrier()
    base = sid * RPT
    pltpu.sync_copy(cacc.at[pl.ds(base, RPT)],
                    cnt_out.at[cid, pl.ds(base, RPT)])

  mesh = plsc.VectorSubcoreMesh(core_axis_name="c", subcore_axis_name="s")
  return pl.kernel(body, out_type=out_type,
                   mesh=mesh, scratch_types=scratch,
                   compiler_params=pltpu.CompilerParams(
                       use_tc_tiling_on_sc=False))


_sc_agg = _build_sc()
_sc_cnt = _build_sc_cnt()


def _proj_body(x_ref, w_ref, b_ref, oa_ref, ob_ref):
  t = jnp.maximum(
      jnp.dot(x_ref[...], w_ref[...], preferred_element_type=jnp.float32)
      + b_ref[...], 0.0)
  oa_ref[...] = t[:, :DH]
  ob_ref[...] = t[:, DH:]


def _proj(x, W, b):
  half = pl.BlockSpec((BM, DH), lambda i: (i, 0))
  return pl.pallas_call(
      _proj_body,
      grid=(N // BM,),
      in_specs=[
          pl.BlockSpec((BM, D), lambda i: (i, 0)),
          pl.BlockSpec((D, D), lambda i: (0, 0)),
          pl.BlockSpec((1, D), lambda i: (0, 0)),
      ],
      out_specs=(half, half),
      out_shape=(jax.ShapeDtypeStruct((N, DH), jnp.float32),
                 jax.ShapeDtypeStruct((N, DH), jnp.float32)),
  )(x, W, b.reshape(1, D))


def _combine(pa, pb, c0, c1, wla, wlb, bl, xr, wr):
  cnt = jnp.maximum(c0[:, 0:1] + c1[:, 0:1], 1.0)
  ma = (pa[0] + pa[1]) / cnt
  mb = (pb[0] + pb[1]) / cnt
  return (jnp.dot(ma, wla[...], preferred_element_type=jnp.float32)
          + jnp.dot(mb, wlb[...], preferred_element_type=jnp.float32)
          + jnp.dot(xr[...], wr[...], preferred_element_type=jnp.float32)
          + bl[...])


def _mid_body(pa, pb, c0, c1, x, wla, wlb, bl, wr, wp2, bp2,
              h_ref, xa_ref, xb_ref):
  t = _combine(pa, pb, c0, c1, wla, wlb, bl, x, wr)
  nrm = jnp.sqrt(jnp.sum(t * t, axis=-1, keepdims=True))
  h = jnp.maximum(t / jnp.maximum(nrm, 1e-12), 0.0)
  h_ref[...] = h
  xp2 = jnp.maximum(
      jnp.dot(h, wp2[...], preferred_element_type=jnp.float32) + bp2[...],
      0.0)
  xa_ref[...] = xp2[:, :DH]
  xb_ref[...] = xp2[:, DH:]


def _mid(pa, pb, c0, c1, x, Wl, bl, Wr, Wp2, bp2):
  row = pl.BlockSpec((BM, D), lambda i: (i, 0))
  part = pl.BlockSpec((NC, BM, DH), lambda i: (0, i, 0))
  cntb = pl.BlockSpec((BM, 16), lambda i: (i, 0))
  whalf = pl.BlockSpec((DH, D), lambda i: (0, 0))
  wspec = pl.BlockSpec((D, D), lambda i: (0, 0))
  bspec = pl.BlockSpec((1, D), lambda i: (0, 0))
  half = pl.BlockSpec((BM, DH), lambda i: (i, 0))
  return pl.pallas_call(
      _mid_body,
      grid=(N // BM,),
      in_specs=[part, part, cntb, cntb, row,
                whalf, whalf, bspec, wspec, wspec, bspec],
      out_specs=(row, half, half),
      out_shape=(jax.ShapeDtypeStruct((N, D), jnp.float32),
                 jax.ShapeDtypeStruct((N, DH), jnp.float32),
                 jax.ShapeDtypeStruct((N, DH), jnp.float32)),
  )(pa, pb, c0, c1, x, Wl[:DH], Wl[DH:],
    bl.reshape(1, D), Wr, Wp2, bp2.reshape(1, D))


def _post_body(pa, pb, c0, c1, h, wla, wlb, bl, wr, o_ref):
  t = _combine(pa, pb, c0, c1, wla, wlb, bl, h, wr)
  nrm = jnp.sqrt(jnp.sum(t * t, axis=-1, keepdims=True))
  o_ref[...] = t / jnp.maximum(nrm, 1e-12)


def _post(pa, pb, c0, c1, h, Wl, bl, Wr):
  row = pl.BlockSpec((BM, D), lambda i: (i, 0))
  part = pl.BlockSpec((NC, BM, DH), lambda i: (0, i, 0))
  cntb = pl.BlockSpec((BM, 16), lambda i: (i, 0))
  whalf = pl.BlockSpec((DH, D), lambda i: (0, 0))
  wspec = pl.BlockSpec((D, D), lambda i: (0, 0))
  bspec = pl.BlockSpec((1, D), lambda i: (0, 0))
  return pl.pallas_call(
      _post_body,
      grid=(N // BM,),
      in_specs=[part, part, cntb, cntb, row,
                whalf, whalf, bspec, wspec],
      out_specs=row,
      out_shape=jax.ShapeDtypeStruct((N, D), jnp.float32),
  )(pa, pb, c0, c1, h, Wl[:DH], Wl[DH:],
    bl.reshape(1, D), Wr)


def kernel(node_emb, edge_index, W1p, b1p, W1l, b1l, W1r,
           W2p, b2p, W2l, b2l, W2r):
  pad = E_PAD - E
  src = jnp.concatenate(
      [edge_index[0], jnp.zeros((pad,), jnp.int32)]).reshape(-1, 128)
  dst = jnp.concatenate(
      [edge_index[1], jnp.full((pad,), N, jnp.int32)]).reshape(-1, 128)
  za = jnp.zeros((RPT, DH), jnp.float32)
  zc = jnp.zeros((RPT, 16), jnp.float32)
  onesc = jnp.ones((128, 16), jnp.float32)

  xa, xb = _proj(node_emb, W1p, b1p)
  cnt1 = _sc_cnt(dst, zc, onesc)
  agg1a, agg1b = _sc_agg(xa, xb, src, dst, za)
  h, x2a, x2b = _mid(agg1a[:, :N], agg1b[:, :N],
                     cnt1[0, :N], cnt1[1, :N],
                     node_emb, W1l, b1l, W1r, W2p, b2p)
  agg2a, agg2b = _sc_agg(x2a, x2b, src, dst, za)
  out = _post(agg2a[:, :N], agg2b[:, :N],
              cnt1[0, :N], cnt1[1, :N], h, W2l, b2l, W2r)
  return out


# speedup vs baseline: 1.2609x; 1.0006x over previous
import functools

import jax
import jax.numpy as jnp
from jax import lax
from jax.experimental import pallas as pl
from jax.experimental.pallas import tpu as pltpu
from jax.experimental.pallas import tpu_sc as plsc

N = 10000
D = 128
DH = 64
E = 320000

NC = 2
NS = 16
NW = NC * NS
E_PAD = 327680
K = 4
CH = K * 128
IRPW = E_PAD // NW // 128
R0 = 152
R1 = 8
NC2_0 = R0 // K // 2
NC2_1 = R1 // K // 2
N_PAD = 10240
RPT = N_PAD // NS

BM = 1000


def _build_sc():
  out_type = (jax.ShapeDtypeStruct((NC, N_PAD, DH), jnp.float32),
              jax.ShapeDtypeStruct((NC, N_PAD, DH), jnp.float32))
  scratch = [
      pltpu.VMEM((K, 128), jnp.int32),
      pltpu.VMEM((K, 128), jnp.int32),
      pltpu.VMEM((K, 128), jnp.int32),
      pltpu.VMEM((K, 128), jnp.int32),
      pltpu.VMEM((CH, DH), jnp.float32),
      pltpu.VMEM((CH, DH), jnp.float32),
      pltpu.VMEM_SHARED((N_PAD, DH), jnp.float32),
      pltpu.SemaphoreType.DMA,
      pltpu.SemaphoreType.DMA,
  ]

  def body(xpa, xpb, src2d, dst2d, za, outa, outb,
           sidx0, didx0, sidx1, didx1, rows0, rows1, acc, sem0, sem1):
    cid = lax.axis_index("c")
    sid = lax.axis_index("s")
    rb0 = jnp.where(cid == 0, sid * R0, NS * R0 + sid * R1)
    nc2 = jnp.where(cid == 0, NC2_0, NC2_1)
    base = sid * RPT

    def phase(xp, out):
      pltpu.sync_copy(za, acc.at[pl.ds(base, RPT)])
      plsc.subcore_barrier()

      def fire(c, si, di, buf, sem):
        rb = rb0 + c * K
        pltpu.sync_copy(src2d.at[pl.ds(rb, K)], si)
        pltpu.sync_copy(dst2d.at[pl.ds(rb, K)], di)

        def g(j, carry):
          pltpu.async_copy(xp.at[si.at[j]],
                           buf.at[pl.ds(j * 128, 128)], sem)
          return carry

        lax.fori_loop(0, K, g, 0)

      def drain(buf, sem):
        pltpu.make_async_copy(xp.at[pl.ds(0, CH)], buf, sem).wait()

      def scatter(di, buf):
        def s(j, carry):
          pltpu.sync_copy(buf.at[pl.ds(j * 128, 128)],
                          acc.at[di.at[j]], add=True)
          return carry

        lax.fori_loop(0, K, s, 0)

      fire(0, sidx0, didx0, rows0, sem0)

      def step(i, carry):
        c0 = 2 * i
        fire(c0 + 1, sidx1, didx1, rows1, sem1)
        drain(rows0, sem0)
        scatter(didx0, rows0)

        @pl.when(i < nc2 - 1)
        def _():
          fire(c0 + 2, sidx0, didx0, rows0, sem0)

        drain(rows1, sem1)
        scatter(didx1, rows1)
        return carry

      lax.fori_loop(0, nc2, step, 0)
      plsc.subcore_barrier()
      pltpu.sync_copy(acc.at[pl.ds(base, RPT)],
                      out.at[cid, pl.ds(base, RPT)])
      plsc.subcore_barrier()

    phase(xpa, outa)
    phase(xpb, outb)

  mesh = plsc.VectorSubcoreMesh(core_axis_name="c", subcore_axis_name="s")
  return pl.kernel(body, out_type=out_type,
                   mesh=mesh, scratch_types=scratch,
                   compiler_params=pltpu.CompilerParams(
                       use_tc_tiling_on_sc=False))


def _build_sc_cnt():
  out_type = jax.ShapeDtypeStruct((NC, N_PAD, 16), jnp.float32)
  scratch = [
      pltpu.VMEM((IRPW, 128), jnp.int32),
      pltpu.VMEM((128, 16), jnp.float32),
      pltpu.VMEM_SHARED((N_PAD, 16), jnp.float32),
  ]

  def body(dst2d, zc, onesc, cnt_out, didx, ones_v, cacc):
    cid = lax.axis_index("c")
    sid = lax.axis_index("s")
    wid = sid * NC + cid
    pltpu.sync_copy(zc, cacc.at[pl.ds(sid * RPT, RPT)])
    pltpu.sync_copy(dst2d.at[pl.ds(wid * IRPW, IRPW)], didx)
    pltpu.sync_copy(onesc, ones_v)
    plsc.subcore_barrier()

    def step(b, carry):
      pltpu.sync_copy(ones_v, cacc.at[didx.at[b]], add=True)
      return carry

    lax.fori_loop(0, IRPW, step, 0)
    plsc.subcore_barrier()
    base = sid * RPT
    pltpu.sync_copy(cacc.at[pl.ds(base, RPT)],
                    cnt_out.at[cid, pl.ds(base, RPT)])

  mesh = plsc.VectorSubcoreMesh(core_axis_name="c", subcore_axis_name="s")
  return pl.kernel(body, out_type=out_type,
                   mesh=mesh, scratch_types=scratch,
                   compiler_params=pltpu.CompilerParams(
                       use_tc_tiling_on_sc=False))


_sc_agg = _build_sc()
_sc_cnt = _build_sc_cnt()


def _proj_body(x_ref, w_ref, b_ref, oa_ref, ob_ref):
  t = jnp.maximum(
      jnp.dot(x_ref[...], w_ref[...], preferred_element_type=jnp.float32)
      + b_ref[...], 0.0)
  oa_ref[...] = t[:, :DH]
  ob_ref[...] = t[:, DH:]


def _proj(x, W, b):
  half = pl.BlockSpec((BM, DH), lambda i: (i, 0))
  return pl.pallas_call(
      _proj_body,
      grid=(N // BM,),
      in_specs=[
          pl.BlockSpec((BM, D), lambda i: (i, 0)),
          pl.BlockSpec((D, D), lambda i: (0, 0)),
          pl.BlockSpec((1, D), lambda i: (0, 0)),
      ],
      out_specs=(half, half),
      out_shape=(jax.ShapeDtypeStruct((N, DH), jnp.float32),
                 jax.ShapeDtypeStruct((N, DH), jnp.float32)),
  )(x, W, b.reshape(1, D))


def _combine(pa, pb, c0, c1, wla, wlb, bl, xr, wr):
  cnt = jnp.maximum(c0[:, 0:1] + c1[:, 0:1], 1.0)
  ma = (pa[0] + pa[1]) / cnt
  mb = (pb[0] + pb[1]) / cnt
  return (jnp.dot(ma, wla[...], preferred_element_type=jnp.float32)
          + jnp.dot(mb, wlb[...], preferred_element_type=jnp.float32)
          + jnp.dot(xr[...], wr[...], preferred_element_type=jnp.float32)
          + bl[...])


def _mid_body(pa, pb, c0, c1, x, wla, wlb, bl, wr, wp2, bp2,
              h_ref, xa_ref, xb_ref):
  t = _combine(pa, pb, c0, c1, wla, wlb, bl, x, wr)
  nrm = jnp.sqrt(jnp.sum(t * t, axis=-1, keepdims=True))
  h = jnp.maximum(t / jnp.maximum(nrm, 1e-12), 0.0)
  h_ref[...] = h
  xp2 = jnp.maximum(
      jnp.dot(h, wp2[...], preferred_element_type=jnp.float32) + bp2[...],
      0.0)
  xa_ref[...] = xp2[:, :DH]
  xb_ref[...] = xp2[:, DH:]


def _mid(pa, pb, c0, c1, x, Wl, bl, Wr, Wp2, bp2):
  row = pl.BlockSpec((BM, D), lambda i: (i, 0))
  part = pl.BlockSpec((NC, BM, DH), lambda i: (0, i, 0))
  cntb = pl.BlockSpec((BM, 16), lambda i: (i, 0))
  whalf = pl.BlockSpec((DH, D), lambda i: (0, 0))
  wspec = pl.BlockSpec((D, D), lambda i: (0, 0))
  bspec = pl.BlockSpec((1, D), lambda i: (0, 0))
  half = pl.BlockSpec((BM, DH), lambda i: (i, 0))
  return pl.pallas_call(
      _mid_body,
      grid=(N // BM,),
      in_specs=[part, part, cntb, cntb, row,
                whalf, whalf, bspec, wspec, wspec, bspec],
      out_specs=(row, half, half),
      out_shape=(jax.ShapeDtypeStruct((N, D), jnp.float32),
                 jax.ShapeDtypeStruct((N, DH), jnp.float32),
                 jax.ShapeDtypeStruct((N, DH), jnp.float32)),
  )(pa, pb, c0, c1, x, Wl[:DH], Wl[DH:],
    bl.reshape(1, D), Wr, Wp2, bp2.reshape(1, D))


def _post_body(pa, pb, c0, c1, h, wla, wlb, bl, wr, o_ref):
  t = _combine(pa, pb, c0, c1, wla, wlb, bl, h, wr)
  nrm = jnp.sqrt(jnp.sum(t * t, axis=-1, keepdims=True))
  o_ref[...] = t / jnp.maximum(nrm, 1e-12)


def _post(pa, pb, c0, c1, h, Wl, bl, Wr):
  row = pl.BlockSpec((BM, D), lambda i: (i, 0))
  part = pl.BlockSpec((NC, BM, DH), lambda i: (0, i, 0))
  cntb = pl.BlockSpec((BM, 16), lambda i: (i, 0))
  whalf = pl.BlockSpec((DH, D), lambda i: (0, 0))
  wspec = pl.BlockSpec((D, D), lambda i: (0, 0))
  bspec = pl.BlockSpec((1, D), lambda i: (0, 0))
  return pl.pallas_call(
      _post_body,
      grid=(N // BM,),
      in_specs=[part, part, cntb, cntb, row,
                whalf, whalf, bspec, wspec],
      out_specs=row,
      out_shape=jax.ShapeDtypeStruct((N, D), jnp.float32),
  )(pa, pb, c0, c1, h, Wl[:DH], Wl[DH:],
    bl.reshape(1, D), Wr)


def kernel(node_emb, edge_index, W1p, b1p, W1l, b1l, W1r,
           W2p, b2p, W2l, b2l, W2r):
  pad = E_PAD - E
  src = jnp.concatenate(
      [edge_index[0], jnp.zeros((pad,), jnp.int32)]).reshape(-1, 128)
  dst = jnp.concatenate(
      [edge_index[1], jnp.full((pad,), N, jnp.int32)]).reshape(-1, 128)
  za = jnp.zeros((RPT, DH), jnp.float32)
  zc = jnp.zeros((RPT, 16), jnp.float32)
  onesc = jnp.ones((128, 16), jnp.float32)

  xa, xb = _proj(node_emb, W1p, b1p)
  cnt1 = _sc_cnt(dst, zc, onesc)
  agg1a, agg1b = _sc_agg(xa, xb, src, dst, za)
  h, x2a, x2b = _mid(agg1a[:, :N], agg1b[:, :N],
                     cnt1[0, :N], cnt1[1, :N],
                     node_emb, W1l, b1l, W1r, W2p, b2p)
  agg2a, agg2b = _sc_agg(x2a, x2b, src, dst, za)
  out = _post(agg2a[:, :N], agg2b[:, :N],
              cnt1[0, :N], cnt1[1, :N], h, W2l, b2l, W2r)
  return out
